# Initial kernel scaffold; baseline (speedup 1.0000x reference)
import jax
import jax.numpy as jnp
from jax.experimental import pallas as pl


def kernel(x, emb_W, A_W, B_W, B_b):
    raise NotImplementedError("write your pallas kernel here")


# speedup vs baseline: 4.4505x; 4.4505x over previous
import functools

import jax
import jax.numpy as jnp
from jax import lax
from jax.experimental import pallas as pl
from jax.experimental.pallas import tpu as pltpu
from jax.experimental.pallas import tpu_sc as plsc

_ROW_BLOCK = 4000
_NC = 2
_NS = 16
_NW = _NC * _NS
_CHUNK = 128


def _fuse_body(emb_ref, a_ref, bt_ref, bias_ref, scale_ref, out_ref):
    acc = jnp.dot(a_ref[...], bt_ref[...], preferred_element_type=jnp.float32)
    out_ref[...] = emb_ref[...] + scale_ref[...] * (acc + bias_ref[...])


def _fused_table(emb_W, A_W, B_Wt, B_b2d, scale):
    num_emb, emb_dim = emb_W.shape
    hidden = A_W.shape[1]
    grid = (num_emb // _ROW_BLOCK,)
    return pl.pallas_call(
        _fuse_body,
        grid=grid,
        in_specs=[
            pl.BlockSpec((_ROW_BLOCK, emb_dim), lambda i: (i, 0)),
            pl.BlockSpec((_ROW_BLOCK, hidden), lambda i: (i, 0)),
            pl.BlockSpec((hidden, emb_dim), lambda i: (0, 0)),
            pl.BlockSpec((1, emb_dim), lambda i: (0, 0)),
            pl.BlockSpec((1, 1), lambda i: (0, 0)),
        ],
        out_specs=pl.BlockSpec((_ROW_BLOCK, emb_dim), lambda i: (i, 0)),
        out_shape=jax.ShapeDtypeStruct((num_emb, emb_dim), jnp.float32),
    )(emb_W, A_W, B_Wt, B_b2d, scale)


@functools.lru_cache(maxsize=None)
def _make_gather(n_tok, emb_dim, num_emb):
    bpw = n_tok // _NW
    nch = bpw // _CHUNK
    mesh = plsc.VectorSubcoreMesh(core_axis_name="c", subcore_axis_name="s")

    @functools.partial(
        pl.kernel,
        mesh=mesh,
        compiler_params=pltpu.CompilerParams(use_tc_tiling_on_sc=False),
        out_type=jax.ShapeDtypeStruct((n_tok, emb_dim), jnp.float32),
        scratch_types=[
            pltpu.VMEM((nch, _CHUNK), jnp.int32),
            pltpu.VMEM((_CHUNK, emb_dim), jnp.float32),
            pltpu.VMEM((_CHUNK, emb_dim), jnp.float32),
            pltpu.SemaphoreType.DMA,
            pltpu.SemaphoreType.DMA,
        ],
    )
    def gather_kernel(table_hbm, idx_hbm, out_hbm,
                      idx_v, rows0, rows1, sem0, sem1):
        wid = lax.axis_index("s") * _NC + lax.axis_index("c")
        base = wid * bpw
        pltpu.sync_copy(idx_hbm.at[wid], idx_v)

        def body(j, _):
            cp = pltpu.async_copy(table_hbm.at[idx_v.at[j]], rows0, sem0)
            cp.wait()
            pltpu.sync_copy(rows0, out_hbm.at[pl.ds(base + j * _CHUNK, _CHUNK)])
            return 0

        lax.fori_loop(0, nch, body, 0, unroll=False)

    return gather_kernel


def kernel(x, emb_W, A_W, B_W, B_b):
    num_emb, emb_dim = emb_W.shape
    hidden = A_W.shape[1]
    scale = jnp.full((1, 1), 0.1 / hidden, dtype=jnp.float32)
    table = _fused_table(emb_W, A_W, B_W.T, B_b.reshape(1, emb_dim), scale)

    n_tok = x.shape[0] * x.shape[1]
    idx = x.reshape(_NW, n_tok // _NW // _CHUNK, _CHUNK)
    out = _make_gather(n_tok, emb_dim, num_emb)(table, idx)
    return out.reshape(x.shape[0], x.shape[1], emb_dim)
